# baseline (speedup 1.0000x reference)
import jax
import jax.numpy as jnp
from jax import lax
from jax.experimental import pallas as pl
from jax.experimental.pallas import tpu as pltpu
from jax.experimental.pallas import tpu_sc as plsc

NUM_ENTITIES = 100000
EMBED_DIM = 128
BATCH = 16384

NC = 2
NS = 16
NW = NC * NS
CHUNK = 64
CHUNKS_PER_W = BATCH // (NW * CHUNK)
ROWS_PER_W = BATCH // NW
NGROUP = CHUNK // 16


def _tec_body(src_hbm, tgt_hbm, rel_hbm, table_hbm, out_hbm,
              idx_v, rel_v, srows0, trows0, srows1, trows1, out_v,
              sem0, sem1):
    cid = lax.axis_index("c")
    sid = lax.axis_index("s")
    wid = sid * NC + cid

    c_rel = pltpu.async_copy(rel_hbm, rel_v, sem0)
    c_src = pltpu.async_copy(src_hbm.at[wid],
                             idx_v.at[pl.ds(0, CHUNKS_PER_W)], sem1)
    c_tgt = pltpu.async_copy(tgt_hbm.at[wid],
                             idx_v.at[pl.ds(CHUNKS_PER_W, CHUNKS_PER_W)],
                             sem1)
    c_rel.wait()
    c_src.wait()
    c_tgt.wait()

    srows = [srows0, srows1]
    trows = [trows0, trows1]
    sems = [sem0, sem1]

    def start(c, b):
        pltpu.async_copy(table_hbm.at[idx_v.at[c]], srows[b], sems[b])
        pltpu.async_copy(table_hbm.at[idx_v.at[CHUNKS_PER_W + c]],
                         trows[b], sems[b])

    def drain(b):
        pltpu.make_async_copy(table_hbm.at[idx_v.at[0]],
                              srows[b], sems[b]).wait()
        pltpu.make_async_copy(table_hbm.at[idx_v.at[0]],
                              trows[b], sems[b]).wait()

    start(0, 0)

    iota16 = lax.iota(jnp.int32, 16)
    rows = [g * 16 + iota16 for g in range(8)]
    zero16 = jnp.zeros((16,), jnp.int32)

    @pl.loop(0, CHUNKS_PER_W, step=2)
    def _chunks(c0):
        for b in range(2):
            c = c0 + b

            @pl.when(c + 1 < CHUNKS_PER_W)
            def _():
                start(c + 1, b ^ 1)

            drain(b)
            sr, tr = srows[b], trows[b]

            def body(col, accs):
                colv = (col + iota16) & (EMBED_DIM - 1)
                r = plsc.load_gather(rel_v, [zero16, colv])
                new = []
                for g in range(NGROUP):
                    s = plsc.load_gather(sr, [rows[g], colv])
                    t = plsc.load_gather(tr, [rows[g], colv])
                    new.append(accs[g] + jnp.abs(s - t + r))
                return tuple(new)

            accs = lax.fori_loop(
                0, EMBED_DIM, body,
                tuple(jnp.zeros((16,), jnp.float32) for _ in range(NGROUP)),
                unroll=1)

            for g in range(NGROUP):
                out_v[pl.ds(c * CHUNK + g * 16, 16)] = accs[g]

    base = pl.multiple_of(wid * ROWS_PER_W, ROWS_PER_W)
    pltpu.sync_copy(out_v, out_hbm.at[pl.ds(base, ROWS_PER_W)])


@jax.jit
def _transe_sc(src3d, tgt3d, rel, table):
    mesh = plsc.VectorSubcoreMesh(core_axis_name="c", subcore_axis_name="s")
    return pl.kernel(
        _tec_body,
        out_type=jax.ShapeDtypeStruct((BATCH,), jnp.float32),
        mesh=mesh,
        compiler_params=pltpu.CompilerParams(
            needs_layout_passes=False,
            skip_device_barrier=True,
            disable_bounds_checks=True,
            disable_semaphore_checks=True,
        ),
        scratch_types=[
            pltpu.VMEM((2 * CHUNKS_PER_W, CHUNK), jnp.int32),
            pltpu.VMEM((1, EMBED_DIM), jnp.float32),
            pltpu.VMEM((CHUNK, EMBED_DIM), jnp.float32),
            pltpu.VMEM((CHUNK, EMBED_DIM), jnp.float32),
            pltpu.VMEM((CHUNK, EMBED_DIM), jnp.float32),
            pltpu.VMEM((CHUNK, EMBED_DIM), jnp.float32),
            pltpu.VMEM((ROWS_PER_W,), jnp.float32),
            pltpu.SemaphoreType.DMA,
            pltpu.SemaphoreType.DMA,
        ],
    )(src3d, tgt3d, rel, table)


def kernel(sources, targets, entity_table, relation_table):
    src3d = sources.astype(jnp.int32).reshape(NW, CHUNKS_PER_W, CHUNK)
    tgt3d = targets.astype(jnp.int32).reshape(NW, CHUNKS_PER_W, CHUNK)
    rel = relation_table.reshape(1, EMBED_DIM).astype(jnp.float32)
    return _transe_sc(src3d, tgt3d, rel, entity_table)


# speedup vs baseline: 1.0260x; 1.0260x over previous
import jax
import jax.numpy as jnp
from jax import lax
from jax.experimental import pallas as pl
from jax.experimental.pallas import tpu as pltpu
from jax.experimental.pallas import tpu_sc as plsc

NUM_ENTITIES = 100000
EMBED_DIM = 128
BATCH = 16384

NC = 2
NS = 16
NW = NC * NS
CHUNK = 128
CHUNKS_PER_W = BATCH // (NW * CHUNK)
ROWS_PER_W = BATCH // NW
NGROUP = CHUNK // 16


def _tec_body(src_hbm, tgt_hbm, rel_hbm, table_hbm, out_hbm,
              idx_v, rel_v, srows, trows, out_v, sems, isem):
    cid = lax.axis_index("c")
    sid = lax.axis_index("s")
    wid = sid * NC + cid

    c_rel = pltpu.async_copy(rel_hbm, rel_v, isem)
    c_src = pltpu.async_copy(src_hbm.at[wid],
                             idx_v.at[pl.ds(0, CHUNKS_PER_W)], isem)
    c_tgt = pltpu.async_copy(tgt_hbm.at[wid],
                             idx_v.at[pl.ds(CHUNKS_PER_W, CHUNKS_PER_W)],
                             isem)
    c_rel.wait()
    c_src.wait()
    c_tgt.wait()

    def start(c, b):
        pltpu.async_copy(table_hbm.at[idx_v.at[c]], srows.at[b], sems.at[b])
        pltpu.async_copy(table_hbm.at[idx_v.at[CHUNKS_PER_W + c]],
                         trows.at[b], sems.at[b])

    start(0, 0)

    iota16 = lax.iota(jnp.int32, 16)
    rows = [g * 16 + iota16 for g in range(NGROUP)]
    zero16 = jnp.zeros((16,), jnp.int32)

    @pl.loop(0, CHUNKS_PER_W)
    def _chunks(c):
        b = c & 1

        @pl.when(c + 1 < CHUNKS_PER_W)
        def _():
            start(c + 1, (c + 1) & 1)

        pltpu.make_async_copy(table_hbm.at[idx_v.at[0]],
                              srows.at[b], sems.at[b]).wait()
        pltpu.make_async_copy(table_hbm.at[idx_v.at[0]],
                              trows.at[b], sems.at[b]).wait()
        sr = srows.at[b]
        tr = trows.at[b]

        def body(col, accs):
            colv = (col + iota16) & (EMBED_DIM - 1)
            r = plsc.load_gather(rel_v, [zero16, colv])
            new = []
            for g in range(NGROUP):
                s = plsc.load_gather(sr, [rows[g], colv])
                t = plsc.load_gather(tr, [rows[g], colv])
                new.append(accs[g] + jnp.abs(s - t + r))
            return tuple(new)

        accs = lax.fori_loop(
            0, EMBED_DIM, body,
            tuple(jnp.zeros((16,), jnp.float32) for _ in range(NGROUP)),
            unroll=1)

        for g in range(NGROUP):
            out_v[pl.ds(c * CHUNK + g * 16, 16)] = accs[g]

    base = pl.multiple_of(wid * ROWS_PER_W, ROWS_PER_W)
    pltpu.sync_copy(out_v, out_hbm.at[pl.ds(base, ROWS_PER_W)])


@jax.jit
def _transe_sc(src3d, tgt3d, rel, table):
    mesh = plsc.VectorSubcoreMesh(core_axis_name="c", subcore_axis_name="s")
    return pl.kernel(
        _tec_body,
        out_type=jax.ShapeDtypeStruct((BATCH,), jnp.float32),
        mesh=mesh,
        compiler_params=pltpu.CompilerParams(
            needs_layout_passes=False,
            skip_device_barrier=True,
            disable_bounds_checks=True,
            disable_semaphore_checks=True,
        ),
        scratch_types=[
            pltpu.VMEM((2 * CHUNKS_PER_W, CHUNK), jnp.int32),
            pltpu.VMEM((1, EMBED_DIM), jnp.float32),
            pltpu.VMEM((2, CHUNK, EMBED_DIM), jnp.float32),
            pltpu.VMEM((2, CHUNK, EMBED_DIM), jnp.float32),
            pltpu.VMEM((ROWS_PER_W,), jnp.float32),
            pltpu.SemaphoreType.DMA((2,)),
            pltpu.SemaphoreType.DMA,
        ],
    )(src3d, tgt3d, rel, table)


def kernel(sources, targets, entity_table, relation_table):
    src3d = sources.astype(jnp.int32).reshape(NW, CHUNKS_PER_W, CHUNK)
    tgt3d = targets.astype(jnp.int32).reshape(NW, CHUNKS_PER_W, CHUNK)
    rel = relation_table.reshape(1, EMBED_DIM).astype(jnp.float32)
    return _transe_sc(src3d, tgt3d, rel, entity_table)
